# Initial kernel scaffold; baseline (speedup 1.0000x reference)
import jax
import jax.numpy as jnp
from jax.experimental import pallas as pl


def kernel(labels, fx, fy, skip0, skip1, edges_nn, params):
    raise NotImplementedError("write your pallas kernel here")


# speedup vs baseline: 3.8307x; 3.8307x over previous
import functools

import jax
import jax.numpy as jnp
from jax import lax
from jax.experimental import pallas as pl
from jax.experimental.pallas import tpu as pltpu
from jax.experimental.pallas import tpu_sc as plsc

B, H, W = 4, 192, 192
HW = H * W
S = 2048
N = B * S
E = 262144
EPS = 1e-5

CP = 208
TW0 = 80
TW1 = 64
PK = 128

@functools.cache
def _mesh():
    return plsc.VectorSubcoreMesh(core_axis_name="c", subcore_axis_name="s")


def _pool_body(pix, lab, zeros, out, rows_v, idx_v, acc):
    c = lax.axis_index("c")
    s = lax.axis_index("s")
    zr = 4096 // 16
    pltpu.sync_copy(zeros.at[pl.ds(s * zr, zr)], acc.at[pl.ds(s * zr, zr)])
    plsc.subcore_barrier()
    per_tile = 2 * HW // 16
    base = c * (2 * HW) + s * per_tile

    def chunk(i, carry):
        off = base + i * PK
        pltpu.sync_copy(pix.at[pl.ds(off, PK)], rows_v)
        pltpu.sync_copy(lab.at[pl.ds(off, PK)], idx_v)
        pltpu.sync_copy(rows_v, acc.at[idx_v], add=True)
        return carry

    lax.fori_loop(0, per_tile // PK, chunk, 0)
    plsc.subcore_barrier()
    pltpu.sync_copy(acc.at[pl.ds(s * zr, zr)],
                    out.at[pl.ds(c * 4096 + s * zr, zr)])


@functools.cache
def _pool_call():
    return pl.kernel(
        _pool_body,
        out_type=jax.ShapeDtypeStruct((N, CP), jnp.float32),
        mesh=_mesh(),
        compiler_params=pltpu.CompilerParams(use_tc_tiling_on_sc=False),
        scratch_types=[
            pltpu.VMEM((PK, CP), jnp.float32),
            pltpu.VMEM((PK,), jnp.int32),
            pltpu.VMEM_SHARED((4096, CP), jnp.float32),
        ],
    )


def _edge0_body(table, src, flat, zeros, out, idx_v, fidx_v, rows_v, sem, acc):
    c = lax.axis_index("c")
    s = lax.axis_index("s")
    zr = (2 * N) // 16
    pltpu.sync_copy(zeros.at[pl.ds(s * zr, zr)], acc.at[pl.ds(s * zr, zr)])
    plsc.subcore_barrier()
    per_tile = E // 32
    base = c * (E // 2) + s * per_tile

    def chunk(i, carry):
        off = base + i * PK
        pltpu.sync_copy(src.at[pl.ds(off, PK)], idx_v)
        pltpu.sync_copy(flat.at[pl.ds(off, PK)], fidx_v)
        pltpu.async_copy(table.at[idx_v], rows_v, sem).wait()
        pltpu.sync_copy(rows_v, acc.at[fidx_v], add=True)
        return carry

    lax.fori_loop(0, per_tile // PK, chunk, 0)
    plsc.subcore_barrier()
    pltpu.sync_copy(acc.at[pl.ds(s * zr, zr)], out.at[c, pl.ds(s * zr, zr)])


@functools.cache
def _edge0_call():
    return pl.kernel(
        _edge0_body,
        out_type=jax.ShapeDtypeStruct((2, 2 * N, TW0), jnp.float32),
        mesh=_mesh(),
        compiler_params=pltpu.CompilerParams(use_tc_tiling_on_sc=False),
        scratch_types=[
            pltpu.VMEM((PK,), jnp.int32),
            pltpu.VMEM((PK,), jnp.int32),
            pltpu.VMEM((PK, TW0), jnp.float32),
            pltpu.SemaphoreType.DMA,
            pltpu.VMEM_SHARED((2 * N, TW0), jnp.float32),
        ],
    )


def _edge1_body(t0, t1, t2, t3, src, flat, zeros,
                o0, o1, o2, o3, idx_v, fidx_v, rows_v, sem, acc):
    c = lax.axis_index("c")
    s = lax.axis_index("s")
    zr = (2 * N) // 16
    per_tile = E // 32
    base = c * (E // 2) + s * per_tile

    for table, out in ((t0, o0), (t1, o1), (t2, o2), (t3, o3)):
        pltpu.sync_copy(zeros.at[pl.ds(s * zr, zr)], acc.at[pl.ds(s * zr, zr)])
        plsc.subcore_barrier()

        def chunk(i, carry, table=table):
            off = base + i * PK
            pltpu.sync_copy(src.at[pl.ds(off, PK)], idx_v)
            pltpu.sync_copy(flat.at[pl.ds(off, PK)], fidx_v)
            pltpu.async_copy(table.at[idx_v], rows_v, sem).wait()
            pltpu.sync_copy(rows_v, acc.at[fidx_v], add=True)
            return carry

        lax.fori_loop(0, per_tile // PK, chunk, 0)
        plsc.subcore_barrier()
        pltpu.sync_copy(acc.at[pl.ds(s * zr, zr)],
                        out.at[c, pl.ds(s * zr, zr)])


@functools.cache
def _edge1_call():
    return pl.kernel(
        _edge1_body,
        out_type=[jax.ShapeDtypeStruct((2, 2 * N, TW1), jnp.float32)] * 4,
        mesh=_mesh(),
        compiler_params=pltpu.CompilerParams(use_tc_tiling_on_sc=False),
        scratch_types=[
            pltpu.VMEM((PK,), jnp.int32),
            pltpu.VMEM((PK,), jnp.int32),
            pltpu.VMEM((PK, TW1), jnp.float32),
            pltpu.SemaphoreType.DMA,
            pltpu.VMEM_SHARED((2 * N, TW1), jnp.float32),
        ],
    )


BM = 1024
GRID = N // BM


def _bn_relu(y, gamma, beta):
    m = jnp.mean(y, axis=0)
    v = jnp.mean((y - m) ** 2, axis=0)
    return jnp.maximum(gamma * (y - m) / jnp.sqrt(v + EPS) + beta, 0.0)


def _mm(a, w):
    return jnp.dot(a, w, preferred_element_type=jnp.float32)


def _row_spec(c):
    return pl.BlockSpec((BM, c), lambda i: (i, 0))


def _full_spec(shape):
    nd = len(shape)
    return pl.BlockSpec(shape, (lambda i: (0,) * nd))


def _tc1_body(pooled_ref, w_ref, b_ref, g_ref, be_ref, out_ref):
    pr = pooled_ref[...]
    feats = pr / pr[:, 196:197]
    y = (_mm(feats[:, 4:68], w_ref[...][:64])
         + _mm(feats[:, 0:4], w_ref[...][64:]) + b_ref[...])
    x0 = _bn_relu(y, g_ref[...], be_ref[...])
    out_ref[...] = jnp.concatenate(
        [x0, jnp.ones((N, 1), jnp.float32), jnp.zeros((N, TW0 - 65), jnp.float32)], axis=1)


def _tc2a_pre_body(ap0, ap1, an0, an1, x0_ref,
                   wpl, wpr, bpr, wnl, wnr, bnr,
                   yp_ref, yn_ref, cnt_ref):
    accp = ap0[...] + ap1[...]
    accn = an0[...] + an1[...]
    cp = jnp.maximum(accp[:, 64:65], 1.0)
    cn = jnp.maximum(accn[:, 64:65], 1.0)
    x0 = x0_ref[...][:, :64]
    yp_ref[...] = _mm(accp[:, :64] / cp, wpl[...]) + _mm(x0, wpr[...]) + bpr[...]
    yn_ref[...] = _mm(accn[:, :64] / cn, wnl[...]) + _mm(x0, wnr[...]) + bnr[...]
    cnt_ref[...] = jnp.concatenate([cp, cn], axis=1)


def _tc2a_bn_body(yp_ref, yn_ref, bng, bnb, x1_ref, x2_ref):
    x1_ref[...] = _bn_relu(yp_ref[...], bng[...][:128], bnb[...][:128])
    x2_ref[...] = _bn_relu(yn_ref[...], bng[...][128:], bnb[...][128:])


def _tc2b_pre_body(x1_ref, x2_ref, pooled_ref, mpw, mpb, mnw, mnb,
                   yp_ref, yn_ref):
    pr = pooled_ref[...]
    cntc = pr[:, 196:197]
    skip1 = pr[:, 68:196] / cntc
    coords = pr[:, 0:4] / cntc
    yp_ref[...] = (_mm(x2_ref[...], mpw[...][:128]) + _mm(skip1, mpw[...][128:256])
                   + _mm(coords, mpw[...][256:]) + mpb[...])
    yn_ref[...] = (_mm(x1_ref[...], mnw[...][:128]) + _mm(skip1, mnw[...][128:256])
                   + _mm(coords, mnw[...][256:]) + mnb[...])


def _tc2b_bn_body(yp_ref, yn_ref, mpg, mpbe, mng, mnbe,
                  xp_lo_ref, xp_hi_ref, xn_lo_ref, xn_hi_ref):
    xp = _bn_relu(yp_ref[...], mpg[...], mpbe[...])
    xn = _bn_relu(yn_ref[...], mng[...], mnbe[...])
    xp_lo_ref[...] = xp[:, :64]
    xp_hi_ref[...] = xp[:, 64:]
    xn_lo_ref[...] = xn[:, :64]
    xn_hi_ref[...] = xn[:, 64:]


def _tc3_pre_body(a0_lo, a1_lo, a0_hi, a1_hi, b0_lo, b1_lo, b0_hi, b1_hi,
                  cnt_ref, xs_lo, xs_hi, wl, wr, bias, y_ref):
    ca = jnp.maximum(cnt_ref[...][:, 0:1], 1.0)
    cb = jnp.maximum(cnt_ref[...][:, 1:2], 1.0)
    y_ref[...] = (_mm((a0_lo[...] + a1_lo[...]) / ca, wl[...][0:64])
                  + _mm((a0_hi[...] + a1_hi[...]) / ca, wl[...][64:128])
                  + _mm((b0_lo[...] + b1_lo[...]) / cb, wl[...][128:192])
                  + _mm((b0_hi[...] + b1_hi[...]) / cb, wl[...][192:256])
                  + _mm(xs_lo[...], wr[...][:64]) + _mm(xs_hi[...], wr[...][64:])
                  + bias[...])


def _tc3_bn_body(y1_ref, y2_ref, bng, bnb, z1_ref, z2_ref):
    z1_ref[...] = _bn_relu(y1_ref[...], bng[...][:256], bnb[...][:256])
    z2_ref[...] = _bn_relu(y2_ref[...], bng[...][256:], bnb[...][256:])


def _tc3b_body(z1_ref, z2_ref, lrw, lrb, out_ref):
    out_ref[...] = jnp.maximum(
        _mm(z1_ref[...], lrw[...][:256]) + _mm(z2_ref[...], lrw[...][256:])
        + lrb[...], 0.0)


def _build_pix(fx, fy, skip0, skip1):
    ii = jnp.arange(H, dtype=jnp.float32) / (H - 1)
    jj = jnp.arange(W, dtype=jnp.float32) / (W - 1)
    xx = jnp.broadcast_to(ii[:, None], (H, W)).reshape(HW, 1)
    yy = jnp.broadcast_to(jj[None, :], (H, W)).reshape(HW, 1)
    coords = jnp.broadcast_to(jnp.concatenate([xx, yy], 1)[None], (B, HW, 2))
    f_x = fx.reshape(B, HW, 1)
    f_y = fy.reshape(B, HW, 1)
    s0 = skip0.reshape(B, 64, HW).transpose(0, 2, 1)
    s1 = skip1.reshape(B, 128, HW).transpose(0, 2, 1)
    ones = jnp.ones((B, HW, 1), jnp.float32)
    pad = jnp.zeros((B, HW, CP - 197), jnp.float32)
    return jnp.concatenate([coords, f_x, f_y, s0, s1, ones, pad], axis=2).reshape(B * HW, CP)


def kernel(labels, fx, fy, skip0, skip1, edges_nn, params):
    f32 = jnp.float32
    pix = _build_pix(fx, fy, skip0, skip1)
    b_half = (jnp.arange(B * HW, dtype=jnp.int32) // HW) & 1
    lab = labels.reshape(B * HW) + b_half * S

    src = edges_nn[0]
    flat = edges_nn[1] + jnp.where(edges_nn[2] == -1, N, 0).astype(jnp.int32)

    pooled = _pool_call()(pix, lab, jnp.zeros((4096, CP), f32))

    p = params['pre_merger']
    x0aug = pl.pallas_call(
        _tc1_body, out_shape=jax.ShapeDtypeStruct((N, TW0), f32),
    )(pooled, p['W'], p['b'], p['gamma'], p['beta'])

    eacc0 = _edge0_call()(x0aug, src, flat, jnp.zeros((2 * N, TW0), f32))

    g = params['gcn0']
    yp, yn, cnt = pl.pallas_call(
        _tc2a_pre_body,
        grid=(GRID,),
        in_specs=[_row_spec(TW0)] * 5
        + [_full_spec(s) for s in ((64, 128), (64, 128), (128,),
                                   (64, 128), (64, 128), (128,))],
        out_specs=[_row_spec(128), _row_spec(128), _row_spec(2)],
        out_shape=[jax.ShapeDtypeStruct((N, 128), f32)] * 2
        + [jax.ShapeDtypeStruct((N, 2), f32)],
    )(eacc0[0, :N], eacc0[1, :N], eacc0[0, N:], eacc0[1, N:], x0aug,
      g['Wpl'], g['Wpr'], g['bpr'], g['Wnl'], g['Wnr'], g['bnr'])
    x1, x2 = pl.pallas_call(
        _tc2a_bn_body,
        out_shape=[jax.ShapeDtypeStruct((N, 128), f32)] * 2,
    )(yp, yn, g['bn_gamma'], g['bn_beta'])

    mp = params['merger_pos0']
    mn = params['merger_neg0']
    yp2, yn2 = pl.pallas_call(
        _tc2b_pre_body,
        grid=(GRID,),
        in_specs=[_row_spec(128), _row_spec(128), _row_spec(CP)]
        + [_full_spec(s) for s in ((260, 128), (128,), (260, 128), (128,))],
        out_specs=[_row_spec(128), _row_spec(128)],
        out_shape=[jax.ShapeDtypeStruct((N, 128), f32)] * 2,
    )(x1, x2, pooled, mp['W'], mp['b'], mn['W'], mn['b'])
    xp_lo, xp_hi, xn_lo, xn_hi = pl.pallas_call(
        _tc2b_bn_body,
        out_shape=[jax.ShapeDtypeStruct((N, 64), f32)] * 4,
    )(yp2, yn2, mp['gamma'], mp['beta'], mn['gamma'], mn['beta'])

    e1 = _edge1_call()(xp_lo, xp_hi, xn_lo, xn_hi, src, flat,
                       jnp.zeros((2 * N, TW1), f32))
    exp_lo, exp_hi, exn_lo, exn_hi = e1

    g1 = params['gcn1']

    def tc3_pre(a_lo, a_hi, b_lo, b_hi, xs_lo, xs_hi, wl, wr, bias):
        return pl.pallas_call(
            _tc3_pre_body,
            grid=(GRID,),
            in_specs=[_row_spec(TW1)] * 8 + [_row_spec(2)]
            + [_row_spec(TW1)] * 2
            + [_full_spec(s) for s in ((256, 256), (128, 256), (256,))],
            out_specs=_row_spec(256),
            out_shape=jax.ShapeDtypeStruct((N, 256), f32),
        )(a_lo[0, :N], a_lo[1, :N], a_hi[0, :N], a_hi[1, :N],
          b_lo[0, N:], b_lo[1, N:], b_hi[0, N:], b_hi[1, N:],
          cnt, xs_lo, xs_hi, wl, wr, bias)

    y1 = tc3_pre(exp_lo, exp_hi, exn_lo, exn_hi, xp_lo, xp_hi,
                 g1['Wpl'], g1['Wpr'], g1['bpr'])
    y2 = tc3_pre(exn_lo, exn_hi, exp_lo, exp_hi, xn_lo, xn_hi,
                 g1['Wnl'], g1['Wnr'], g1['bnr'])
    z1, z2 = pl.pallas_call(
        _tc3_bn_body,
        out_shape=[jax.ShapeDtypeStruct((N, 256), f32)] * 2,
    )(y1, y2, g1['bn_gamma'], g1['bn_beta'])

    lr = params['lin_reduc']
    return pl.pallas_call(
        _tc3b_body, out_shape=jax.ShapeDtypeStruct((N, 256), f32),
    )(z1, z2, lr['W'], lr['b'])


# speedup vs baseline: 6.6954x; 1.7478x over previous
import functools

import jax
import jax.numpy as jnp
from jax import lax
from jax.experimental import pallas as pl
from jax.experimental.pallas import tpu as pltpu
from jax.experimental.pallas import tpu_sc as plsc

B, H, W = 4, 192, 192
HW = H * W
S = 2048
N = B * S
E = 262144
EPS = 1e-5

TW0 = 80
TW1 = 64
PK = 128

@functools.cache
def _mesh():
    return plsc.VectorSubcoreMesh(core_axis_name="c", subcore_axis_name="s")


def _zero_zbuf(zbuf, cols):
    z16 = jnp.zeros((16,), jnp.float32)

    def row(r, carry):
        for k in range(cols // 16):
            zbuf[r, pl.ds(k * 16, 16)] = z16
        return carry

    lax.fori_loop(0, PK, row, 0)


def _zero_acc(zbuf, acc, rows, cols, row0):
    for k in range(rows // PK):
        pltpu.sync_copy(zbuf.at[:, pl.ds(0, cols)],
                        acc.at[pl.ds(row0 + k * PK, PK), pl.ds(0, cols)])


def _pool_body(small8, s0t, s1t, lab, o8, o64, o128,
               ib0, ib1, r8_0, r8_1, r64_0, r64_1, r128_0, r128_1,
               sem0, sem1, zbuf, a8, a64, a128):
    c = lax.axis_index("c")
    s = lax.axis_index("s")
    zr = 4096 // 16
    _zero_zbuf(zbuf, 128)
    _zero_acc(zbuf, a8, zr, 8, s * zr)
    _zero_acc(zbuf, a64, zr, 64, s * zr)
    _zero_acc(zbuf, a128, zr, 128, s * zr)
    plsc.subcore_barrier()

    nc = 2 * HW // 16 // PK
    tb = (c * 2 * HW + s * (2 * HW // 16)) // PK
    ibufs = (ib0, ib1)
    r8s, r64s, r128s = (r8_0, r8_1), (r64_0, r64_1), (r128_0, r128_1)
    sems = (sem0, sem1)

    def load(ci, b):
        off = (tb + ci) * PK
        pltpu.sync_copy(lab.at[pl.ds(off, PK)], ibufs[b])
        pltpu.async_copy(small8.at[pl.ds(off, PK)], r8s[b], sems[b])
        pltpu.async_copy(s0t.at[pl.ds(off, PK)], r64s[b], sems[b])
        pltpu.async_copy(s1t.at[pl.ds(off, PK)], r128s[b], sems[b])

    def drain(b):
        pltpu.make_async_copy(small8.at[pl.ds(0, PK)], r8s[b], sems[b]).wait()
        pltpu.make_async_copy(s0t.at[pl.ds(0, PK)], r64s[b], sems[b]).wait()
        pltpu.make_async_copy(s1t.at[pl.ds(0, PK)], r128s[b], sems[b]).wait()

    load(0, 0)

    def pair(i2, carry):
        for b in (0, 1):
            ci = 2 * i2 + b

            @pl.when(ci + 1 < nc)
            def _():
                load(ci + 1, 1 - b)

            drain(b)
            pltpu.sync_copy(r8s[b], a8.at[ibufs[b]], add=True)
            pltpu.sync_copy(r64s[b], a64.at[ibufs[b]], add=True)
            pltpu.sync_copy(r128s[b], a128.at[ibufs[b]], add=True)
        return carry

    lax.fori_loop(0, nc // 2, pair, 0)
    plsc.subcore_barrier()
    pltpu.sync_copy(a8.at[pl.ds(s * zr, zr)],
                    o8.at[pl.ds(c * 4096 + s * zr, zr)])
    pltpu.sync_copy(a64.at[pl.ds(s * zr, zr)],
                    o64.at[pl.ds(c * 4096 + s * zr, zr)])
    pltpu.sync_copy(a128.at[pl.ds(s * zr, zr)],
                    o128.at[pl.ds(c * 4096 + s * zr, zr)])


@functools.cache
def _pool_call():
    f32 = jnp.float32
    return pl.kernel(
        _pool_body,
        out_type=[jax.ShapeDtypeStruct((N, 8), f32),
                  jax.ShapeDtypeStruct((N, 64), f32),
                  jax.ShapeDtypeStruct((N, 128), f32)],
        mesh=_mesh(),
        compiler_params=pltpu.CompilerParams(use_tc_tiling_on_sc=False),
        scratch_types=[
            pltpu.VMEM((PK,), jnp.int32),
            pltpu.VMEM((PK,), jnp.int32),
            pltpu.VMEM((PK, 8), f32),
            pltpu.VMEM((PK, 8), f32),
            pltpu.VMEM((PK, 64), f32),
            pltpu.VMEM((PK, 64), f32),
            pltpu.VMEM((PK, 128), f32),
            pltpu.VMEM((PK, 128), f32),
            pltpu.SemaphoreType.DMA,
            pltpu.SemaphoreType.DMA,
            pltpu.VMEM((PK, 128), f32),
            pltpu.VMEM_SHARED((4096, 8), f32),
            pltpu.VMEM_SHARED((4096, 64), f32),
            pltpu.VMEM_SHARED((4096, 128), f32),
        ],
    )


def _edge_pass(table, edges, out, bufs, acc, zbuf, c, s, zr, width):
    _zero_zbuf(zbuf, width)
    _zero_acc(zbuf, acc, zr, width, s * zr)
    plsc.subcore_barrier()
    nc = E // 32 // PK
    tb = c * (E // 2 // PK) + s * nc
    sbufs, fbufs, rbufs, gbufs, sems = bufs

    def load(ci, b):
        off = (tb + ci) * PK
        pltpu.sync_copy(edges.at[0, pl.ds(off, PK)], sbufs[b])
        pltpu.sync_copy(edges.at[1, pl.ds(off, PK)], fbufs[b])
        pltpu.sync_copy(edges.at[2, pl.ds(off, PK)], gbufs[b])
        for j in range(PK // 16):
            d16 = fbufs[b][pl.ds(j * 16, 16)]
            g16 = gbufs[b][pl.ds(j * 16, 16)]
            fbufs[b][pl.ds(j * 16, 16)] = d16 + jnp.where(
                g16 < 0, jnp.int32(N), jnp.int32(0))
        pltpu.async_copy(table.at[sbufs[b]], rbufs[b], sems[b])

    load(0, 0)

    def pair(i2, carry):
        for b in (0, 1):
            ci = 2 * i2 + b

            @pl.when(ci + 1 < nc)
            def _():
                load(ci + 1, 1 - b)

            pltpu.make_async_copy(table.at[sbufs[b]], rbufs[b], sems[b]).wait()
            pltpu.sync_copy(rbufs[b], acc.at[fbufs[b]], add=True)
        return carry

    lax.fori_loop(0, nc // 2, pair, 0)
    plsc.subcore_barrier()
    pltpu.sync_copy(acc.at[pl.ds(s * zr, zr)], out.at[c, pl.ds(s * zr, zr)])


def _edge_scratch(width):
    f32 = jnp.float32
    return [
        pltpu.VMEM((PK,), jnp.int32),
        pltpu.VMEM((PK,), jnp.int32),
        pltpu.VMEM((PK,), jnp.int32),
        pltpu.VMEM((PK,), jnp.int32),
        pltpu.VMEM((PK,), jnp.int32),
        pltpu.VMEM((PK,), jnp.int32),
        pltpu.VMEM((PK, width), f32),
        pltpu.VMEM((PK, width), f32),
        pltpu.SemaphoreType.DMA,
        pltpu.SemaphoreType.DMA,
        pltpu.VMEM((PK, width), f32),
        pltpu.VMEM_SHARED((2 * N, width), f32),
    ]


def _edge0_body(table, edges, out, s0, s1, f0, f1, g0, g1, r0, r1,
                sem0, sem1, zbuf, acc):
    c = lax.axis_index("c")
    s = lax.axis_index("s")
    bufs = ((s0, s1), (f0, f1), (r0, r1), (g0, g1), (sem0, sem1))
    _edge_pass(table, edges, out, bufs, acc, zbuf, c, s, (2 * N) // 16, TW0)


@functools.cache
def _edge0_call():
    return pl.kernel(
        _edge0_body,
        out_type=jax.ShapeDtypeStruct((2, 2 * N, TW0), jnp.float32),
        mesh=_mesh(),
        compiler_params=pltpu.CompilerParams(use_tc_tiling_on_sc=False),
        scratch_types=_edge_scratch(TW0),
    )


def _edge1_body(t0, t1, t2, t3, edges, o0, o1, o2, o3,
                s0, s1, f0, f1, g0, g1, r0, r1, sem0, sem1, zbuf, acc):
    c = lax.axis_index("c")
    s = lax.axis_index("s")
    bufs = ((s0, s1), (f0, f1), (r0, r1), (g0, g1), (sem0, sem1))
    for table, out in ((t0, o0), (t1, o1), (t2, o2), (t3, o3)):
        _edge_pass(table, edges, out, bufs, acc, zbuf, c, s,
                   (2 * N) // 16, TW1)


@functools.cache
def _edge1_call():
    return pl.kernel(
        _edge1_body,
        out_type=[jax.ShapeDtypeStruct((2, 2 * N, TW1), jnp.float32)] * 4,
        mesh=_mesh(),
        compiler_params=pltpu.CompilerParams(use_tc_tiling_on_sc=False),
        scratch_types=_edge_scratch(TW1),
    )


BM = 1024
GRID = N // BM


def _bn_relu(y, gamma, beta):
    m = jnp.mean(y, axis=0)
    v = jnp.mean((y - m) ** 2, axis=0)
    return jnp.maximum(gamma * (y - m) / jnp.sqrt(v + EPS) + beta, 0.0)


def _mm(a, w):
    return jnp.dot(a, w, preferred_element_type=jnp.float32)


def _row_spec(c):
    return pl.BlockSpec((BM, c), lambda i: (i, 0))


def _acc_spec(c, row_block_off):
    return pl.BlockSpec((2, BM, c), lambda i: (0, i + row_block_off, 0))


def _full_spec(shape):
    nd = len(shape)
    return pl.BlockSpec(shape, (lambda i: (0,) * nd))


def _tc1_pre_body(p8_ref, p64_ref, w_ref, b_ref, y_ref):
    p8 = p8_ref[...]
    cnt = p8[:, 4:5]
    y_ref[...] = (_mm(p64_ref[...] / cnt, w_ref[...][:64])
                  + _mm(p8[:, 0:4] / cnt, w_ref[...][64:]) + b_ref[...])


def _tc1_bn_body(y_ref, g_ref, be_ref, out_ref):
    x0 = _bn_relu(y_ref[...], g_ref[...], be_ref[...])
    out_ref[...] = jnp.concatenate(
        [x0, jnp.ones((N, 1), jnp.float32), jnp.zeros((N, TW0 - 65), jnp.float32)], axis=1)


def _tc2a_pre_body(ap_ref, an_ref, x0_ref,
                   wpl, wpr, bpr, wnl, wnr, bnr,
                   yp_ref, yn_ref, cnt_ref):
    accp = ap_ref[0] + ap_ref[1]
    accn = an_ref[0] + an_ref[1]
    cp = jnp.maximum(accp[:, 64:65], 1.0)
    cn = jnp.maximum(accn[:, 64:65], 1.0)
    x0 = x0_ref[...][:, :64]
    yp_ref[...] = _mm(accp[:, :64] / cp, wpl[...]) + _mm(x0, wpr[...]) + bpr[...]
    yn_ref[...] = _mm(accn[:, :64] / cn, wnl[...]) + _mm(x0, wnr[...]) + bnr[...]
    cnt_ref[...] = jnp.concatenate([cp, cn], axis=1)


def _tc2a_bn_body(yp_ref, yn_ref, bng, bnb, x1_ref, x2_ref):
    x1_ref[...] = _bn_relu(yp_ref[...], bng[...][:128], bnb[...][:128])
    x2_ref[...] = _bn_relu(yn_ref[...], bng[...][128:], bnb[...][128:])


def _tc2b_pre_body(x1_ref, x2_ref, p8_ref, p128_ref, mpw, mpb, mnw, mnb,
                   yp_ref, yn_ref):
    p8 = p8_ref[...]
    cnt = p8[:, 4:5]
    skip1 = p128_ref[...] / cnt
    coords = p8[:, 0:4] / cnt
    yp_ref[...] = (_mm(x2_ref[...], mpw[...][:128]) + _mm(skip1, mpw[...][128:256])
                   + _mm(coords, mpw[...][256:]) + mpb[...])
    yn_ref[...] = (_mm(x1_ref[...], mnw[...][:128]) + _mm(skip1, mnw[...][128:256])
                   + _mm(coords, mnw[...][256:]) + mnb[...])


def _tc2b_bn_body(yp_ref, yn_ref, mpg, mpbe, mng, mnbe,
                  xp_lo_ref, xp_hi_ref, xn_lo_ref, xn_hi_ref):
    xp = _bn_relu(yp_ref[...], mpg[...], mpbe[...])
    xn = _bn_relu(yn_ref[...], mng[...], mnbe[...])
    xp_lo_ref[...] = xp[:, :64]
    xp_hi_ref[...] = xp[:, 64:]
    xn_lo_ref[...] = xn[:, :64]
    xn_hi_ref[...] = xn[:, 64:]


def _tc3_pre_body(a_lo, a_hi, b_lo, b_hi,
                  cnt_ref, xs_lo, xs_hi, wl, wr, bias, y_ref):
    ca = jnp.maximum(cnt_ref[...][:, 0:1], 1.0)
    cb = jnp.maximum(cnt_ref[...][:, 1:2], 1.0)
    y_ref[...] = (_mm((a_lo[0] + a_lo[1]) / ca, wl[...][0:64])
                  + _mm((a_hi[0] + a_hi[1]) / ca, wl[...][64:128])
                  + _mm((b_lo[0] + b_lo[1]) / cb, wl[...][128:192])
                  + _mm((b_hi[0] + b_hi[1]) / cb, wl[...][192:256])
                  + _mm(xs_lo[...], wr[...][:64]) + _mm(xs_hi[...], wr[...][64:])
                  + bias[...])


def _tc3_bn_body(y1_ref, y2_ref, bng, bnb, z1_ref, z2_ref):
    z1_ref[...] = _bn_relu(y1_ref[...], bng[...][:256], bnb[...][:256])
    z2_ref[...] = _bn_relu(y2_ref[...], bng[...][256:], bnb[...][256:])


def _tc3b_body(z1_ref, z2_ref, lrw, lrb, out_ref):
    out_ref[...] = jnp.maximum(
        _mm(z1_ref[...], lrw[...][:256]) + _mm(z2_ref[...], lrw[...][256:])
        + lrb[...], 0.0)


def kernel(labels, fx, fy, skip0, skip1, edges_nn, params):
    f32 = jnp.float32
    ii = jnp.arange(H, dtype=f32) / (H - 1)
    jj = jnp.arange(W, dtype=f32) / (W - 1)
    xx = jnp.broadcast_to(ii[:, None], (H, W)).reshape(HW, 1)
    yy = jnp.broadcast_to(jj[None, :], (H, W)).reshape(HW, 1)
    coords = jnp.broadcast_to(jnp.concatenate([xx, yy], 1)[None], (B, HW, 2))
    small8 = jnp.concatenate(
        [coords, fx.reshape(B, HW, 1), fy.reshape(B, HW, 1),
         jnp.ones((B, HW, 1), f32), jnp.zeros((B, HW, 3), f32)],
        axis=2).reshape(B * HW, 8)
    s0t = skip0.reshape(B, 64, HW).transpose(0, 2, 1).reshape(B * HW, 64)
    s1t = skip1.reshape(B, 128, HW).transpose(0, 2, 1).reshape(B * HW, 128)
    b_half = (jnp.arange(B * HW, dtype=jnp.int32) // HW) & 1
    lab = labels.reshape(B * HW) + b_half * S

    p8, p64, p128 = _pool_call()(small8, s0t, s1t, lab)

    p = params['pre_merger']
    y0 = pl.pallas_call(
        _tc1_pre_body,
        grid=(GRID,),
        in_specs=[_row_spec(8), _row_spec(64),
                  _full_spec((68, 64)), _full_spec((64,))],
        out_specs=_row_spec(64),
        out_shape=jax.ShapeDtypeStruct((N, 64), f32),
    )(p8, p64, p['W'], p['b'])
    x0aug = pl.pallas_call(
        _tc1_bn_body, out_shape=jax.ShapeDtypeStruct((N, TW0), f32),
    )(y0, p['gamma'], p['beta'])

    eacc0 = _edge0_call()(x0aug, edges_nn)

    g = params['gcn0']
    yp, yn, cnt = pl.pallas_call(
        _tc2a_pre_body,
        grid=(GRID,),
        in_specs=[_acc_spec(TW0, 0), _acc_spec(TW0, GRID), _row_spec(TW0)]
        + [_full_spec(s) for s in ((64, 128), (64, 128), (128,),
                                   (64, 128), (64, 128), (128,))],
        out_specs=[_row_spec(128), _row_spec(128), _row_spec(2)],
        out_shape=[jax.ShapeDtypeStruct((N, 128), f32)] * 2
        + [jax.ShapeDtypeStruct((N, 2), f32)],
    )(eacc0, eacc0, x0aug,
      g['Wpl'], g['Wpr'], g['bpr'], g['Wnl'], g['Wnr'], g['bnr'])
    x1, x2 = pl.pallas_call(
        _tc2a_bn_body,
        out_shape=[jax.ShapeDtypeStruct((N, 128), f32)] * 2,
    )(yp, yn, g['bn_gamma'], g['bn_beta'])

    mp = params['merger_pos0']
    mn = params['merger_neg0']
    yp2, yn2 = pl.pallas_call(
        _tc2b_pre_body,
        grid=(GRID,),
        in_specs=[_row_spec(128), _row_spec(128), _row_spec(8), _row_spec(128)]
        + [_full_spec(s) for s in ((260, 128), (128,), (260, 128), (128,))],
        out_specs=[_row_spec(128), _row_spec(128)],
        out_shape=[jax.ShapeDtypeStruct((N, 128), f32)] * 2,
    )(x1, x2, p8, p128, mp['W'], mp['b'], mn['W'], mn['b'])
    xp_lo, xp_hi, xn_lo, xn_hi = pl.pallas_call(
        _tc2b_bn_body,
        out_shape=[jax.ShapeDtypeStruct((N, 64), f32)] * 4,
    )(yp2, yn2, mp['gamma'], mp['beta'], mn['gamma'], mn['beta'])

    e1 = _edge1_call()(xp_lo, xp_hi, xn_lo, xn_hi, edges_nn)
    exp_lo, exp_hi, exn_lo, exn_hi = e1

    g1 = params['gcn1']

    def tc3_pre(a_lo, a_hi, b_lo, b_hi, xs_lo, xs_hi, wl, wr, bias):
        return pl.pallas_call(
            _tc3_pre_body,
            grid=(GRID,),
            in_specs=[_acc_spec(TW1, 0), _acc_spec(TW1, 0),
                      _acc_spec(TW1, GRID), _acc_spec(TW1, GRID), _row_spec(2)]
            + [_row_spec(TW1)] * 2
            + [_full_spec(s) for s in ((256, 256), (128, 256), (256,))],
            out_specs=_row_spec(256),
            out_shape=jax.ShapeDtypeStruct((N, 256), f32),
        )(a_lo, a_hi, b_lo, b_hi, cnt, xs_lo, xs_hi, wl, wr, bias)

    y1 = tc3_pre(exp_lo, exp_hi, exn_lo, exn_hi, xp_lo, xp_hi,
                 g1['Wpl'], g1['Wpr'], g1['bpr'])
    y2 = tc3_pre(exn_lo, exn_hi, exp_lo, exp_hi, xn_lo, xn_hi,
                 g1['Wnl'], g1['Wnr'], g1['bnr'])
    z1, z2 = pl.pallas_call(
        _tc3_bn_body,
        out_shape=[jax.ShapeDtypeStruct((N, 256), f32)] * 2,
    )(y1, y2, g1['bn_gamma'], g1['bn_beta'])

    lr = params['lin_reduc']
    return pl.pallas_call(
        _tc3b_body, out_shape=jax.ShapeDtypeStruct((N, 256), f32),
    )(z1, z2, lr['W'], lr['b'])


# speedup vs baseline: 8.7825x; 1.3117x over previous
import functools

import jax
import jax.numpy as jnp
from jax import lax
from jax.experimental import pallas as pl
from jax.experimental.pallas import tpu as pltpu
from jax.experimental.pallas import tpu_sc as plsc

B, H, W = 4, 192, 192
HW = H * W
S = 2048
N = B * S
E = 262144
EPS = 1e-5

TW0 = 80
TW1 = 64
PK = 128

@functools.cache
def _mesh():
    return plsc.VectorSubcoreMesh(core_axis_name="c", subcore_axis_name="s")


def _zero_zbuf(zbuf, cols):
    z16 = jnp.zeros((16,), jnp.float32)

    def row(r, carry):
        for k in range(cols // 16):
            zbuf[r, pl.ds(k * 16, 16)] = z16
        return carry

    lax.fori_loop(0, PK, row, 0)


def _zero_acc(zbuf, acc, rows, cols, row0):
    for k in range(rows // PK):
        pltpu.sync_copy(zbuf.at[:, pl.ds(0, cols)],
                        acc.at[pl.ds(row0 + k * PK, PK), pl.ds(0, cols)])


def _pool_body(small8, s0t, s1t, lab, o8, o64, o128,
               ib0, ib1, r8_0, r8_1, r64_0, r64_1, r128_0, r128_1,
               sem0, sem1, zbuf, a8, a64, a128):
    c = lax.axis_index("c")
    s = lax.axis_index("s")
    zr = 4096 // 16
    _zero_zbuf(zbuf, 128)
    _zero_acc(zbuf, a8, zr, 8, s * zr)
    _zero_acc(zbuf, a64, zr, 64, s * zr)
    _zero_acc(zbuf, a128, zr, 128, s * zr)
    plsc.subcore_barrier()

    nc = 2 * HW // 16 // PK
    tb = (c * 2 * HW + s * (2 * HW // 16)) // PK
    ibufs = (ib0, ib1)
    r8s, r64s, r128s = (r8_0, r8_1), (r64_0, r64_1), (r128_0, r128_1)
    sems = (sem0, sem1)

    def load(ci, b):
        off = (tb + ci) * PK
        pltpu.sync_copy(lab.at[pl.ds(off, PK)], ibufs[b])
        pltpu.async_copy(small8.at[pl.ds(off, PK)], r8s[b], sems[b])
        pltpu.async_copy(s0t.at[pl.ds(off, PK)], r64s[b], sems[b])
        pltpu.async_copy(s1t.at[pl.ds(off, PK)], r128s[b], sems[b])

    def drain(b):
        pltpu.make_async_copy(small8.at[pl.ds(0, PK)], r8s[b], sems[b]).wait()
        pltpu.make_async_copy(s0t.at[pl.ds(0, PK)], r64s[b], sems[b]).wait()
        pltpu.make_async_copy(s1t.at[pl.ds(0, PK)], r128s[b], sems[b]).wait()

    load(0, 0)

    def pair(i2, carry):
        for b in (0, 1):
            ci = 2 * i2 + b

            @pl.when(ci + 1 < nc)
            def _():
                load(ci + 1, 1 - b)

            drain(b)
            pltpu.sync_copy(r8s[b], a8.at[ibufs[b]], add=True)
            pltpu.sync_copy(r64s[b], a64.at[ibufs[b]], add=True)
            pltpu.sync_copy(r128s[b], a128.at[ibufs[b]], add=True)
        return carry

    lax.fori_loop(0, nc // 2, pair, 0)
    plsc.subcore_barrier()
    pltpu.sync_copy(a8.at[pl.ds(s * zr, zr)],
                    o8.at[pl.ds(c * 4096 + s * zr, zr)])
    pltpu.sync_copy(a64.at[pl.ds(s * zr, zr)],
                    o64.at[pl.ds(c * 4096 + s * zr, zr)])
    pltpu.sync_copy(a128.at[pl.ds(s * zr, zr)],
                    o128.at[pl.ds(c * 4096 + s * zr, zr)])


@functools.cache
def _pool_call():
    f32 = jnp.float32
    return pl.kernel(
        _pool_body,
        out_type=[jax.ShapeDtypeStruct((N, 8), f32),
                  jax.ShapeDtypeStruct((N, 64), f32),
                  jax.ShapeDtypeStruct((N, 128), f32)],
        mesh=_mesh(),
        compiler_params=pltpu.CompilerParams(use_tc_tiling_on_sc=False),
        scratch_types=[
            pltpu.VMEM((PK,), jnp.int32),
            pltpu.VMEM((PK,), jnp.int32),
            pltpu.VMEM((PK, 8), f32),
            pltpu.VMEM((PK, 8), f32),
            pltpu.VMEM((PK, 64), f32),
            pltpu.VMEM((PK, 64), f32),
            pltpu.VMEM((PK, 128), f32),
            pltpu.VMEM((PK, 128), f32),
            pltpu.SemaphoreType.DMA,
            pltpu.SemaphoreType.DMA,
            pltpu.VMEM((PK, 128), f32),
            pltpu.VMEM_SHARED((4096, 8), f32),
            pltpu.VMEM_SHARED((4096, 64), f32),
            pltpu.VMEM_SHARED((4096, 128), f32),
        ],
    )


def _edge_pass(table, ixf, out, bufs, acc, zbuf, c, s, zr, width):
    _zero_zbuf(zbuf, width)
    _zero_acc(zbuf, acc, zr, width, s * zr)
    plsc.subcore_barrier()
    nc = E // 32 // PK
    tb = c * (E // 2 // PK) + s * nc
    ibufs, rbufs, sems = bufs

    def load(ci, b):
        pltpu.sync_copy(ixf.at[tb + ci], ibufs[b])
        pltpu.async_copy(table.at[ibufs[b].at[0]], rbufs[b], sems[b])

    load(0, 0)

    def pair(i2, carry):
        for b in (0, 1):
            ci = 2 * i2 + b

            @pl.when(ci + 1 < nc)
            def _():
                load(ci + 1, 1 - b)

            pltpu.make_async_copy(table.at[ibufs[b].at[0]],
                                  rbufs[b], sems[b]).wait()
            pltpu.sync_copy(rbufs[b], acc.at[ibufs[b].at[1]], add=True)
        return carry

    lax.fori_loop(0, nc // 2, pair, 0)
    plsc.subcore_barrier()
    pltpu.sync_copy(acc.at[pl.ds(s * zr, zr)], out.at[c, pl.ds(s * zr, zr)])


def _edge_scratch(width):
    f32 = jnp.float32
    return [
        pltpu.VMEM((2, PK), jnp.int32),
        pltpu.VMEM((2, PK), jnp.int32),
        pltpu.VMEM((PK, width), f32),
        pltpu.VMEM((PK, width), f32),
        pltpu.SemaphoreType.DMA,
        pltpu.SemaphoreType.DMA,
        pltpu.VMEM((PK, width), f32),
        pltpu.VMEM_SHARED((2 * N, width), f32),
    ]


def _edge0_body(table, ixf, out, i0, i1, r0, r1, sem0, sem1, zbuf, acc):
    c = lax.axis_index("c")
    s = lax.axis_index("s")
    bufs = ((i0, i1), (r0, r1), (sem0, sem1))
    _edge_pass(table, ixf, out, bufs, acc, zbuf, c, s, (2 * N) // 16, TW0)


@functools.cache
def _edge0_call():
    return pl.kernel(
        _edge0_body,
        out_type=jax.ShapeDtypeStruct((2, 2 * N, TW0), jnp.float32),
        mesh=_mesh(),
        compiler_params=pltpu.CompilerParams(use_tc_tiling_on_sc=False),
        scratch_types=_edge_scratch(TW0),
    )


def _edge1_body(t0, t1, t2, t3, ixf, o0, o1, o2, o3,
                i0, i1, r0, r1, sem0, sem1, zbuf, acc):
    c = lax.axis_index("c")
    s = lax.axis_index("s")
    bufs = ((i0, i1), (r0, r1), (sem0, sem1))
    for table, out in ((t0, o0), (t1, o1), (t2, o2), (t3, o3)):
        _edge_pass(table, ixf, out, bufs, acc, zbuf, c, s,
                   (2 * N) // 16, TW1)


@functools.cache
def _edge1_call():
    return pl.kernel(
        _edge1_body,
        out_type=[jax.ShapeDtypeStruct((2, 2 * N, TW1), jnp.float32)] * 4,
        mesh=_mesh(),
        compiler_params=pltpu.CompilerParams(use_tc_tiling_on_sc=False),
        scratch_types=_edge_scratch(TW1),
    )


BM = 1024
GRID = N // BM


def _bn_relu(y, gamma, beta):
    m = jnp.mean(y, axis=0)
    v = jnp.mean((y - m) ** 2, axis=0)
    return jnp.maximum(gamma * (y - m) / jnp.sqrt(v + EPS) + beta, 0.0)


def _mm(a, w):
    return jnp.dot(a, w, preferred_element_type=jnp.float32)


def _row_spec(c):
    return pl.BlockSpec((BM, c), lambda i: (i, 0))


def _acc_spec(c, row_block_off):
    return pl.BlockSpec((2, BM, c), lambda i: (0, i + row_block_off, 0))


def _full_spec(shape):
    nd = len(shape)
    return pl.BlockSpec(shape, (lambda i: (0,) * nd))


def _tc1_pre_body(p8_ref, p64_ref, w_ref, b_ref, y_ref):
    p8 = p8_ref[...]
    cnt = p8[:, 4:5]
    y_ref[...] = (_mm(p64_ref[...] / cnt, w_ref[...][:64])
                  + _mm(p8[:, 0:4] / cnt, w_ref[...][64:]) + b_ref[...])


def _tc1_bn_body(y_ref, g_ref, be_ref, out_ref):
    x0 = _bn_relu(y_ref[...], g_ref[...], be_ref[...])
    out_ref[...] = jnp.concatenate(
        [x0, jnp.ones((N, 1), jnp.float32), jnp.zeros((N, TW0 - 65), jnp.float32)], axis=1)


def _tc2a_pre_body(ap_ref, an_ref, x0_ref,
                   wpl, wpr, bpr, wnl, wnr, bnr,
                   yp_ref, yn_ref, cnt_ref):
    accp = ap_ref[0] + ap_ref[1]
    accn = an_ref[0] + an_ref[1]
    cp = jnp.maximum(accp[:, 64:65], 1.0)
    cn = jnp.maximum(accn[:, 64:65], 1.0)
    x0 = x0_ref[...][:, :64]
    yp_ref[...] = _mm(accp[:, :64] / cp, wpl[...]) + _mm(x0, wpr[...]) + bpr[...]
    yn_ref[...] = _mm(accn[:, :64] / cn, wnl[...]) + _mm(x0, wnr[...]) + bnr[...]
    cnt_ref[...] = jnp.concatenate([cp, cn], axis=1)


def _tc2a_bn_body(yp_ref, yn_ref, bng, bnb, x1_ref, x2_ref):
    x1_ref[...] = _bn_relu(yp_ref[...], bng[...][:128], bnb[...][:128])
    x2_ref[...] = _bn_relu(yn_ref[...], bng[...][128:], bnb[...][128:])


def _tc2b_pre_body(x1_ref, x2_ref, p8_ref, p128_ref, mpw, mpb, mnw, mnb,
                   yp_ref, yn_ref):
    p8 = p8_ref[...]
    cnt = p8[:, 4:5]
    skip1 = p128_ref[...] / cnt
    coords = p8[:, 0:4] / cnt
    yp_ref[...] = (_mm(x2_ref[...], mpw[...][:128]) + _mm(skip1, mpw[...][128:256])
                   + _mm(coords, mpw[...][256:]) + mpb[...])
    yn_ref[...] = (_mm(x1_ref[...], mnw[...][:128]) + _mm(skip1, mnw[...][128:256])
                   + _mm(coords, mnw[...][256:]) + mnb[...])


def _tc2b_bn_body(yp_ref, yn_ref, mpg, mpbe, mng, mnbe,
                  xp_lo_ref, xp_hi_ref, xn_lo_ref, xn_hi_ref):
    xp = _bn_relu(yp_ref[...], mpg[...], mpbe[...])
    xn = _bn_relu(yn_ref[...], mng[...], mnbe[...])
    xp_lo_ref[...] = xp[:, :64]
    xp_hi_ref[...] = xp[:, 64:]
    xn_lo_ref[...] = xn[:, :64]
    xn_hi_ref[...] = xn[:, 64:]


def _tc3_pre_body(a_lo, a_hi, b_lo, b_hi,
                  cnt_ref, xs_lo, xs_hi, wl, wr, bias, y_ref):
    ca = jnp.maximum(cnt_ref[...][:, 0:1], 1.0)
    cb = jnp.maximum(cnt_ref[...][:, 1:2], 1.0)
    y_ref[...] = (_mm((a_lo[0] + a_lo[1]) / ca, wl[...][0:64])
                  + _mm((a_hi[0] + a_hi[1]) / ca, wl[...][64:128])
                  + _mm((b_lo[0] + b_lo[1]) / cb, wl[...][128:192])
                  + _mm((b_hi[0] + b_hi[1]) / cb, wl[...][192:256])
                  + _mm(xs_lo[...], wr[...][:64]) + _mm(xs_hi[...], wr[...][64:])
                  + bias[...])


def _tc3_bn_body(y1_ref, y2_ref, bng, bnb, z1_ref, z2_ref):
    z1_ref[...] = _bn_relu(y1_ref[...], bng[...][:256], bnb[...][:256])
    z2_ref[...] = _bn_relu(y2_ref[...], bng[...][256:], bnb[...][256:])


def _tc3b_body(z1_ref, z2_ref, lrw, lrb, out_ref):
    out_ref[...] = jnp.maximum(
        _mm(z1_ref[...], lrw[...][:256]) + _mm(z2_ref[...], lrw[...][256:])
        + lrb[...], 0.0)


def kernel(labels, fx, fy, skip0, skip1, edges_nn, params):
    f32 = jnp.float32
    ii = jnp.arange(H, dtype=f32) / (H - 1)
    jj = jnp.arange(W, dtype=f32) / (W - 1)
    xx = jnp.broadcast_to(ii[:, None], (H, W)).reshape(HW, 1)
    yy = jnp.broadcast_to(jj[None, :], (H, W)).reshape(HW, 1)
    coords = jnp.broadcast_to(jnp.concatenate([xx, yy], 1)[None], (B, HW, 2))
    small8 = jnp.concatenate(
        [coords, fx.reshape(B, HW, 1), fy.reshape(B, HW, 1),
         jnp.ones((B, HW, 1), f32), jnp.zeros((B, HW, 3), f32)],
        axis=2).reshape(B * HW, 8)
    s0t = skip0.reshape(B, 64, HW).transpose(0, 2, 1).reshape(B * HW, 64)
    s1t = skip1.reshape(B, 128, HW).transpose(0, 2, 1).reshape(B * HW, 128)
    b_half = (jnp.arange(B * HW, dtype=jnp.int32) // HW) & 1
    lab = labels.reshape(B * HW) + b_half * S
    flat = edges_nn[1] + jnp.where(edges_nn[2] < 0, N, 0).astype(jnp.int32)
    ixf = jnp.stack([edges_nn[0].reshape(E // PK, PK),
                     flat.reshape(E // PK, PK)], axis=1)

    p8, p64, p128 = _pool_call()(small8, s0t, s1t, lab)

    p = params['pre_merger']
    y0 = pl.pallas_call(
        _tc1_pre_body,
        grid=(GRID,),
        in_specs=[_row_spec(8), _row_spec(64),
                  _full_spec((68, 64)), _full_spec((64,))],
        out_specs=_row_spec(64),
        out_shape=jax.ShapeDtypeStruct((N, 64), f32),
    )(p8, p64, p['W'], p['b'])
    x0aug = pl.pallas_call(
        _tc1_bn_body, out_shape=jax.ShapeDtypeStruct((N, TW0), f32),
    )(y0, p['gamma'], p['beta'])

    eacc0 = _edge0_call()(x0aug, ixf)

    g = params['gcn0']
    yp, yn, cnt = pl.pallas_call(
        _tc2a_pre_body,
        grid=(GRID,),
        in_specs=[_acc_spec(TW0, 0), _acc_spec(TW0, GRID), _row_spec(TW0)]
        + [_full_spec(s) for s in ((64, 128), (64, 128), (128,),
                                   (64, 128), (64, 128), (128,))],
        out_specs=[_row_spec(128), _row_spec(128), _row_spec(2)],
        out_shape=[jax.ShapeDtypeStruct((N, 128), f32)] * 2
        + [jax.ShapeDtypeStruct((N, 2), f32)],
    )(eacc0, eacc0, x0aug,
      g['Wpl'], g['Wpr'], g['bpr'], g['Wnl'], g['Wnr'], g['bnr'])
    x1, x2 = pl.pallas_call(
        _tc2a_bn_body,
        out_shape=[jax.ShapeDtypeStruct((N, 128), f32)] * 2,
    )(yp, yn, g['bn_gamma'], g['bn_beta'])

    mp = params['merger_pos0']
    mn = params['merger_neg0']
    yp2, yn2 = pl.pallas_call(
        _tc2b_pre_body,
        grid=(GRID,),
        in_specs=[_row_spec(128), _row_spec(128), _row_spec(8), _row_spec(128)]
        + [_full_spec(s) for s in ((260, 128), (128,), (260, 128), (128,))],
        out_specs=[_row_spec(128), _row_spec(128)],
        out_shape=[jax.ShapeDtypeStruct((N, 128), f32)] * 2,
    )(x1, x2, p8, p128, mp['W'], mp['b'], mn['W'], mn['b'])
    xp_lo, xp_hi, xn_lo, xn_hi = pl.pallas_call(
        _tc2b_bn_body,
        out_shape=[jax.ShapeDtypeStruct((N, 64), f32)] * 4,
    )(yp2, yn2, mp['gamma'], mp['beta'], mn['gamma'], mn['beta'])

    e1 = _edge1_call()(xp_lo, xp_hi, xn_lo, xn_hi, ixf)
    exp_lo, exp_hi, exn_lo, exn_hi = e1

    g1 = params['gcn1']

    def tc3_pre(a_lo, a_hi, b_lo, b_hi, xs_lo, xs_hi, wl, wr, bias):
        return pl.pallas_call(
            _tc3_pre_body,
            grid=(GRID,),
            in_specs=[_acc_spec(TW1, 0), _acc_spec(TW1, 0),
                      _acc_spec(TW1, GRID), _acc_spec(TW1, GRID), _row_spec(2)]
            + [_row_spec(TW1)] * 2
            + [_full_spec(s) for s in ((256, 256), (128, 256), (256,))],
            out_specs=_row_spec(256),
            out_shape=jax.ShapeDtypeStruct((N, 256), f32),
        )(a_lo, a_hi, b_lo, b_hi, cnt, xs_lo, xs_hi, wl, wr, bias)

    y1 = tc3_pre(exp_lo, exp_hi, exn_lo, exn_hi, xp_lo, xp_hi,
                 g1['Wpl'], g1['Wpr'], g1['bpr'])
    y2 = tc3_pre(exn_lo, exn_hi, exp_lo, exp_hi, xn_lo, xn_hi,
                 g1['Wnl'], g1['Wnr'], g1['bnr'])
    z1, z2 = pl.pallas_call(
        _tc3_bn_body,
        out_shape=[jax.ShapeDtypeStruct((N, 256), f32)] * 2,
    )(y1, y2, g1['bn_gamma'], g1['bn_beta'])

    lr = params['lin_reduc']
    return pl.pallas_call(
        _tc3b_body, out_shape=jax.ShapeDtypeStruct((N, 256), f32),
    )(z1, z2, lr['W'], lr['b'])


# speedup vs baseline: 9.6426x; 1.0979x over previous
import functools

import jax
import jax.numpy as jnp
from jax import lax
from jax.experimental import pallas as pl
from jax.experimental.pallas import tpu as pltpu
from jax.experimental.pallas import tpu_sc as plsc

B, H, W = 4, 192, 192
HW = H * W
S = 2048
N = B * S
E = 262144
EPS = 1e-5

TW0 = 80
TW1 = 64
PK = 128

@functools.cache
def _mesh():
    return plsc.VectorSubcoreMesh(core_axis_name="c", subcore_axis_name="s")


def _zero_zbuf(zbuf, cols):
    z16 = jnp.zeros((16,), jnp.float32)

    def row(r, carry):
        for k in range(cols // 16):
            zbuf[r, pl.ds(k * 16, 16)] = z16
        return carry

    lax.fori_loop(0, PK, row, 0)


def _zero_acc(zbuf, acc, rows, cols, row0):
    for k in range(rows // PK):
        pltpu.sync_copy(zbuf.at[:, pl.ds(0, cols)],
                        acc.at[pl.ds(row0 + k * PK, PK), pl.ds(0, cols)])


def _pool_body(small8, s0t, s1t, lab, o8, o64, o128,
               ib0, ib1, r8_0, r8_1, r64_0, r64_1, r128_0, r128_1,
               sem0, sem1, a8, a64, a128):
    c = lax.axis_index("c")
    s = lax.axis_index("s")
    zr = 4096 // 16
    _zero_zbuf(r128_0, 128)
    _zero_acc(r128_0, a8, zr, 8, s * zr)
    _zero_acc(r128_0, a64, zr, 64, s * zr)
    _zero_acc(r128_0, a128, zr, 128, s * zr)
    plsc.subcore_barrier()

    nc = 2 * HW // 16 // PK
    tb = (c * 2 * HW + s * (2 * HW // 16)) // PK
    ibufs = (ib0, ib1)
    r8s, r64s, r128s = (r8_0, r8_1), (r64_0, r64_1), (r128_0, r128_1)
    sems = (sem0, sem1)

    def load(ci, b):
        off = (tb + ci) * PK
        pltpu.async_copy(lab.at[pl.ds(off, PK)], ibufs[b], sems[b])
        pltpu.async_copy(small8.at[pl.ds(off, PK)], r8s[b], sems[b])
        pltpu.async_copy(s0t.at[pl.ds(off, PK)], r64s[b], sems[b])
        pltpu.async_copy(s1t.at[pl.ds(off, PK)], r128s[b], sems[b])

    def drain(b):
        pltpu.make_async_copy(lab.at[pl.ds(0, PK)], ibufs[b], sems[b]).wait()
        pltpu.make_async_copy(small8.at[pl.ds(0, PK)], r8s[b], sems[b]).wait()
        pltpu.make_async_copy(s0t.at[pl.ds(0, PK)], r64s[b], sems[b]).wait()
        pltpu.make_async_copy(s1t.at[pl.ds(0, PK)], r128s[b], sems[b]).wait()

    load(0, 0)

    def pair(i2, carry):
        for b in (0, 1):
            ci = 2 * i2 + b

            @pl.when(ci + 1 < nc)
            def _():
                load(ci + 1, 1 - b)

            drain(b)
            pltpu.sync_copy(r8s[b], a8.at[ibufs[b]], add=True)
            pltpu.sync_copy(r64s[b], a64.at[ibufs[b]], add=True)
            pltpu.sync_copy(r128s[b], a128.at[ibufs[b]], add=True)
        return carry

    lax.fori_loop(0, nc // 2, pair, 0)
    plsc.subcore_barrier()
    pltpu.sync_copy(a8.at[pl.ds(s * zr, zr)],
                    o8.at[pl.ds(c * 4096 + s * zr, zr)])
    pltpu.sync_copy(a64.at[pl.ds(s * zr, zr)],
                    o64.at[pl.ds(c * 4096 + s * zr, zr)])
    pltpu.sync_copy(a128.at[pl.ds(s * zr, zr)],
                    o128.at[pl.ds(c * 4096 + s * zr, zr)])


@functools.cache
def _pool_call():
    f32 = jnp.float32
    return pl.kernel(
        _pool_body,
        out_type=[jax.ShapeDtypeStruct((N, 8), f32),
                  jax.ShapeDtypeStruct((N, 64), f32),
                  jax.ShapeDtypeStruct((N, 128), f32)],
        mesh=_mesh(),
        compiler_params=pltpu.CompilerParams(use_tc_tiling_on_sc=False),
        scratch_types=(
            [pltpu.VMEM((PK,), jnp.int32)] * 2
            + [pltpu.VMEM((PK, 8), f32)] * 2
            + [pltpu.VMEM((PK, 64), f32)] * 2
            + [pltpu.VMEM((PK, 128), f32)] * 2
            + [pltpu.SemaphoreType.DMA] * 2
            + [pltpu.VMEM_SHARED((4096, 8), f32),
               pltpu.VMEM_SHARED((4096, 64), f32),
               pltpu.VMEM_SHARED((4096, 128), f32)]
        ),
    )


NB = 8


def _edge_pass(table, ixf4, out, bufs, acc, c, s, zr, width, rd):
    ibufs, isems, rbufs, rsems = bufs
    _zero_zbuf(rbufs[0], width)
    _zero_acc(rbufs[0], acc, zr, width, s * zr)
    plsc.subcore_barrier()
    nc = E // 32 // PK
    nblk = nc // NB
    tbb = c * (E // 2 // PK // NB) + s * nblk

    pltpu.sync_copy(ixf4.at[tbb], ibufs[0])
    pltpu.async_copy(ixf4.at[tbb + 1], ibufs[1], isems[1])
    for k0 in range(rd - 1):
        pltpu.async_copy(table.at[ibufs[0].at[k0, 0]], rbufs[k0], rsems[k0])

    def pairblock(j2, carry):
        for jj in (0, 1):
            j = 2 * j2 + jj
            ibc, ibn = ibufs[jj], ibufs[1 - jj]
            for k in range(NB):
                ci = j * NB + k
                nk, gslot = k + rd - 1, (k + rd - 1) % rd
                if nk == NB:
                    @pl.when(j < nblk - 1)
                    def _():
                        pltpu.make_async_copy(
                            ixf4.at[tbb], ibn, isems[1 - jj]).wait()
                slot = k % rd
                pltpu.make_async_copy(table.at[ibc.at[k, 0]],
                                      rbufs[slot], rsems[slot]).wait()
                pltpu.sync_copy(rbufs[slot], acc.at[ibc.at[k, 1]], add=True)
                if nk < NB:
                    @pl.when(ci + rd - 1 < nc)
                    def _(nk=nk, gslot=gslot, ibc=ibc):
                        pltpu.async_copy(table.at[ibc.at[nk, 0]],
                                         rbufs[gslot], rsems[gslot])
                else:
                    @pl.when(ci + rd - 1 < nc)
                    def _(nk=nk - NB, gslot=gslot, ibn=ibn):
                        pltpu.async_copy(table.at[ibn.at[nk, 0]],
                                         rbufs[gslot], rsems[gslot])
                if k == NB - 1:
                    @pl.when(j < nblk - 2)
                    def _(ibc=ibc, jj=jj):
                        pltpu.async_copy(ixf4.at[tbb + j + 2], ibc, isems[jj])
        return carry

    lax.fori_loop(0, nblk // 2, pairblock, 0)
    plsc.subcore_barrier()
    pltpu.sync_copy(acc.at[pl.ds(s * zr, zr)], out.at[c, pl.ds(s * zr, zr)])


def _edge_scratch(width, rd):
    f32 = jnp.float32
    return (
        [pltpu.VMEM((NB, 2, PK), jnp.int32)] * 2
        + [pltpu.SemaphoreType.DMA] * 2
        + [pltpu.VMEM((PK, width), f32)] * rd
        + [pltpu.SemaphoreType.DMA] * rd
        + [pltpu.VMEM_SHARED((2 * N, width), f32)]
    )


def _edge0_body(table, ixf4, out, i0, i1, is0, is1,
                r0, r1, rs0, rs1, acc):
    c = lax.axis_index("c")
    s = lax.axis_index("s")
    bufs = ((i0, i1), (is0, is1), (r0, r1), (rs0, rs1))
    _edge_pass(table, ixf4, out, bufs, acc, c, s, (2 * N) // 16, TW0, 2)


@functools.cache
def _edge0_call():
    return pl.kernel(
        _edge0_body,
        out_type=jax.ShapeDtypeStruct((2, 2 * N, TW0), jnp.float32),
        mesh=_mesh(),
        compiler_params=pltpu.CompilerParams(use_tc_tiling_on_sc=False),
        scratch_types=_edge_scratch(TW0, 2),
    )


def _edge1_body(t0, t1, t2, t3, ixf4, o0, o1, o2, o3, i0, i1, is0, is1,
                r0, r1, r2, r3, rs0, rs1, rs2, rs3, acc):
    c = lax.axis_index("c")
    s = lax.axis_index("s")
    bufs = ((i0, i1), (is0, is1), (r0, r1, r2, r3), (rs0, rs1, rs2, rs3))
    for table, out in ((t0, o0), (t1, o1), (t2, o2), (t3, o3)):
        _edge_pass(table, ixf4, out, bufs, acc, c, s, (2 * N) // 16, TW1, 4)


@functools.cache
def _edge1_call():
    return pl.kernel(
        _edge1_body,
        out_type=[jax.ShapeDtypeStruct((2, 2 * N, TW1), jnp.float32)] * 4,
        mesh=_mesh(),
        compiler_params=pltpu.CompilerParams(use_tc_tiling_on_sc=False),
        scratch_types=_edge_scratch(TW1, 4),
    )


BM = 1024
GRID = N // BM


def _bn_relu(y, gamma, beta):
    m = jnp.mean(y, axis=0)
    v = jnp.mean((y - m) ** 2, axis=0)
    return jnp.maximum(gamma * (y - m) / jnp.sqrt(v + EPS) + beta, 0.0)


def _mm(a, w):
    return jnp.dot(a, w, preferred_element_type=jnp.float32)


def _row_spec(c):
    return pl.BlockSpec((BM, c), lambda i: (i, 0))


def _acc_spec(c, row_block_off):
    return pl.BlockSpec((2, BM, c), lambda i: (0, i + row_block_off, 0))


def _full_spec(shape):
    nd = len(shape)
    return pl.BlockSpec(shape, (lambda i: (0,) * nd))


def _tc1_pre_body(p8_ref, p64_ref, w_ref, b_ref, y_ref):
    p8 = p8_ref[...]
    cnt = p8[:, 4:5]
    y_ref[...] = (_mm(p64_ref[...] / cnt, w_ref[...][:64])
                  + _mm(p8[:, 0:4] / cnt, w_ref[...][64:]) + b_ref[...])


def _tc1_bn_body(y_ref, g_ref, be_ref, out_ref):
    x0 = _bn_relu(y_ref[...], g_ref[...], be_ref[...])
    out_ref[...] = jnp.concatenate(
        [x0, jnp.ones((N, 1), jnp.float32), jnp.zeros((N, TW0 - 65), jnp.float32)], axis=1)


def _tc2a_pre_body(ap_ref, an_ref, x0_ref,
                   wpl, wpr, bpr, wnl, wnr, bnr,
                   yp_ref, yn_ref, cnt_ref):
    accp = ap_ref[0] + ap_ref[1]
    accn = an_ref[0] + an_ref[1]
    cp = jnp.maximum(accp[:, 64:65], 1.0)
    cn = jnp.maximum(accn[:, 64:65], 1.0)
    x0 = x0_ref[...][:, :64]
    yp_ref[...] = _mm(accp[:, :64] / cp, wpl[...]) + _mm(x0, wpr[...]) + bpr[...]
    yn_ref[...] = _mm(accn[:, :64] / cn, wnl[...]) + _mm(x0, wnr[...]) + bnr[...]
    cnt_ref[...] = jnp.concatenate([cp, cn], axis=1)


def _tc2a_bn_body(yp_ref, yn_ref, bng, bnb, x1_ref, x2_ref):
    x1_ref[...] = _bn_relu(yp_ref[...], bng[...][:128], bnb[...][:128])
    x2_ref[...] = _bn_relu(yn_ref[...], bng[...][128:], bnb[...][128:])


def _tc2b_pre_body(x1_ref, x2_ref, p8_ref, p128_ref, mpw, mpb, mnw, mnb,
                   yp_ref, yn_ref):
    p8 = p8_ref[...]
    cnt = p8[:, 4:5]
    skip1 = p128_ref[...] / cnt
    coords = p8[:, 0:4] / cnt
    yp_ref[...] = (_mm(x2_ref[...], mpw[...][:128]) + _mm(skip1, mpw[...][128:256])
                   + _mm(coords, mpw[...][256:]) + mpb[...])
    yn_ref[...] = (_mm(x1_ref[...], mnw[...][:128]) + _mm(skip1, mnw[...][128:256])
                   + _mm(coords, mnw[...][256:]) + mnb[...])


def _tc2b_bn_body(yp_ref, yn_ref, mpg, mpbe, mng, mnbe,
                  xp_lo_ref, xp_hi_ref, xn_lo_ref, xn_hi_ref):
    xp = _bn_relu(yp_ref[...], mpg[...], mpbe[...])
    xn = _bn_relu(yn_ref[...], mng[...], mnbe[...])
    xp_lo_ref[...] = xp[:, :64]
    xp_hi_ref[...] = xp[:, 64:]
    xn_lo_ref[...] = xn[:, :64]
    xn_hi_ref[...] = xn[:, 64:]


def _tc3_pre_body(a_lo, a_hi, b_lo, b_hi,
                  cnt_ref, xs_lo, xs_hi, wl, wr, bias, y_ref):
    ca = jnp.maximum(cnt_ref[...][:, 0:1], 1.0)
    cb = jnp.maximum(cnt_ref[...][:, 1:2], 1.0)
    y_ref[...] = (_mm((a_lo[0] + a_lo[1]) / ca, wl[...][0:64])
                  + _mm((a_hi[0] + a_hi[1]) / ca, wl[...][64:128])
                  + _mm((b_lo[0] + b_lo[1]) / cb, wl[...][128:192])
                  + _mm((b_hi[0] + b_hi[1]) / cb, wl[...][192:256])
                  + _mm(xs_lo[...], wr[...][:64]) + _mm(xs_hi[...], wr[...][64:])
                  + bias[...])


def _tc3_bn_body(y1_ref, y2_ref, bng, bnb, z1_ref, z2_ref):
    z1_ref[...] = _bn_relu(y1_ref[...], bng[...][:256], bnb[...][:256])
    z2_ref[...] = _bn_relu(y2_ref[...], bng[...][256:], bnb[...][256:])


def _tc3b_body(z1_ref, z2_ref, lrw, lrb, out_ref):
    out_ref[...] = jnp.maximum(
        _mm(z1_ref[...], lrw[...][:256]) + _mm(z2_ref[...], lrw[...][256:])
        + lrb[...], 0.0)


def kernel(labels, fx, fy, skip0, skip1, edges_nn, params):
    f32 = jnp.float32
    ii = jnp.arange(H, dtype=f32) / (H - 1)
    jj = jnp.arange(W, dtype=f32) / (W - 1)
    xx = jnp.broadcast_to(ii[:, None], (H, W)).reshape(HW, 1)
    yy = jnp.broadcast_to(jj[None, :], (H, W)).reshape(HW, 1)
    coords = jnp.broadcast_to(jnp.concatenate([xx, yy], 1)[None], (B, HW, 2))
    small8 = jnp.concatenate(
        [coords, fx.reshape(B, HW, 1), fy.reshape(B, HW, 1),
         jnp.ones((B, HW, 1), f32), jnp.zeros((B, HW, 3), f32)],
        axis=2).reshape(B * HW, 8)
    s0t = skip0.reshape(B, 64, HW).transpose(0, 2, 1).reshape(B * HW, 64)
    s1t = skip1.reshape(B, 128, HW).transpose(0, 2, 1).reshape(B * HW, 128)
    b_half = (jnp.arange(B * HW, dtype=jnp.int32) // HW) & 1
    lab = labels.reshape(B * HW) + b_half * S
    flat = edges_nn[1] + jnp.where(edges_nn[2] < 0, N, 0).astype(jnp.int32)
    ixf = jnp.stack([edges_nn[0].reshape(E // PK, PK),
                     flat.reshape(E // PK, PK)],
                    axis=1).reshape(E // PK // NB, NB, 2, PK)

    p8, p64, p128 = _pool_call()(small8, s0t, s1t, lab)

    p = params['pre_merger']
    y0 = pl.pallas_call(
        _tc1_pre_body,
        grid=(GRID,),
        in_specs=[_row_spec(8), _row_spec(64),
                  _full_spec((68, 64)), _full_spec((64,))],
        out_specs=_row_spec(64),
        out_shape=jax.ShapeDtypeStruct((N, 64), f32),
    )(p8, p64, p['W'], p['b'])
    x0aug = pl.pallas_call(
        _tc1_bn_body, out_shape=jax.ShapeDtypeStruct((N, TW0), f32),
    )(y0, p['gamma'], p['beta'])

    eacc0 = _edge0_call()(x0aug, ixf)

    g = params['gcn0']
    yp, yn, cnt = pl.pallas_call(
        _tc2a_pre_body,
        grid=(GRID,),
        in_specs=[_acc_spec(TW0, 0), _acc_spec(TW0, GRID), _row_spec(TW0)]
        + [_full_spec(s) for s in ((64, 128), (64, 128), (128,),
                                   (64, 128), (64, 128), (128,))],
        out_specs=[_row_spec(128), _row_spec(128), _row_spec(2)],
        out_shape=[jax.ShapeDtypeStruct((N, 128), f32)] * 2
        + [jax.ShapeDtypeStruct((N, 2), f32)],
    )(eacc0, eacc0, x0aug,
      g['Wpl'], g['Wpr'], g['bpr'], g['Wnl'], g['Wnr'], g['bnr'])
    x1, x2 = pl.pallas_call(
        _tc2a_bn_body,
        out_shape=[jax.ShapeDtypeStruct((N, 128), f32)] * 2,
    )(yp, yn, g['bn_gamma'], g['bn_beta'])

    mp = params['merger_pos0']
    mn = params['merger_neg0']
    yp2, yn2 = pl.pallas_call(
        _tc2b_pre_body,
        grid=(GRID,),
        in_specs=[_row_spec(128), _row_spec(128), _row_spec(8), _row_spec(128)]
        + [_full_spec(s) for s in ((260, 128), (128,), (260, 128), (128,))],
        out_specs=[_row_spec(128), _row_spec(128)],
        out_shape=[jax.ShapeDtypeStruct((N, 128), f32)] * 2,
    )(x1, x2, p8, p128, mp['W'], mp['b'], mn['W'], mn['b'])
    xp_lo, xp_hi, xn_lo, xn_hi = pl.pallas_call(
        _tc2b_bn_body,
        out_shape=[jax.ShapeDtypeStruct((N, 64), f32)] * 4,
    )(yp2, yn2, mp['gamma'], mp['beta'], mn['gamma'], mn['beta'])

    e1 = _edge1_call()(xp_lo, xp_hi, xn_lo, xn_hi, ixf)
    exp_lo, exp_hi, exn_lo, exn_hi = e1

    g1 = params['gcn1']

    def tc3_pre(a_lo, a_hi, b_lo, b_hi, xs_lo, xs_hi, wl, wr, bias):
        return pl.pallas_call(
            _tc3_pre_body,
            grid=(GRID,),
            in_specs=[_acc_spec(TW1, 0), _acc_spec(TW1, 0),
                      _acc_spec(TW1, GRID), _acc_spec(TW1, GRID), _row_spec(2)]
            + [_row_spec(TW1)] * 2
            + [_full_spec(s) for s in ((256, 256), (128, 256), (256,))],
            out_specs=_row_spec(256),
            out_shape=jax.ShapeDtypeStruct((N, 256), f32),
        )(a_lo, a_hi, b_lo, b_hi, cnt, xs_lo, xs_hi, wl, wr, bias)

    y1 = tc3_pre(exp_lo, exp_hi, exn_lo, exn_hi, xp_lo, xp_hi,
                 g1['Wpl'], g1['Wpr'], g1['bpr'])
    y2 = tc3_pre(exn_lo, exn_hi, exp_lo, exp_hi, xn_lo, xn_hi,
                 g1['Wnl'], g1['Wnr'], g1['bnr'])
    z1, z2 = pl.pallas_call(
        _tc3_bn_body,
        out_shape=[jax.ShapeDtypeStruct((N, 256), f32)] * 2,
    )(y1, y2, g1['bn_gamma'], g1['bn_beta'])

    lr = params['lin_reduc']
    return pl.pallas_call(
        _tc3b_body, out_shape=jax.ShapeDtypeStruct((N, 256), f32),
    )(z1, z2, lr['W'], lr['b'])


# speedup vs baseline: 10.3347x; 1.0718x over previous
import functools

import jax
import jax.numpy as jnp
from jax import lax
from jax.experimental import pallas as pl
from jax.experimental.pallas import tpu as pltpu
from jax.experimental.pallas import tpu_sc as plsc

B, H, W = 4, 192, 192
HW = H * W
S = 2048
N = B * S
E = 262144
EPS = 1e-5

TW0 = 80
TW1 = 64
PK = 128

@functools.cache
def _mesh():
    return plsc.VectorSubcoreMesh(core_axis_name="c", subcore_axis_name="s")


def _zero_zbuf(zbuf, cols):
    z16 = jnp.zeros((16,), jnp.float32)

    def row(r, carry):
        for k in range(cols // 16):
            zbuf[r, pl.ds(k * 16, 16)] = z16
        return carry

    lax.fori_loop(0, PK, row, 0)


def _zero_acc(zbuf, acc, rows, cols, row0):
    for k in range(rows // PK):
        pltpu.sync_copy(zbuf.at[:, pl.ds(0, cols)],
                        acc.at[pl.ds(row0 + k * PK, PK), pl.ds(0, cols)])


def _pool_body(small8, s0t, s1t, lab, o8, o64, o128,
               ib0, ib1, r8_0, r8_1, r64_0, r64_1, r128_0, r128_1,
               sem0, sem1, a8, a64, a128):
    c = lax.axis_index("c")
    s = lax.axis_index("s")
    zr = 4096 // 16
    _zero_zbuf(r128_0, 128)
    _zero_acc(r128_0, a8, zr, 8, s * zr)
    _zero_acc(r128_0, a64, zr, 64, s * zr)
    _zero_acc(r128_0, a128, zr, 128, s * zr)
    plsc.subcore_barrier()

    nc = 2 * HW // 16 // PK
    tb = (c * 2 * HW + s * (2 * HW // 16)) // PK
    ibufs = (ib0, ib1)
    r8s, r64s, r128s = (r8_0, r8_1), (r64_0, r64_1), (r128_0, r128_1)
    sems = (sem0, sem1)

    def load(ci, b):
        off = (tb + ci) * PK
        pltpu.async_copy(lab.at[pl.ds(off, PK)], ibufs[b], sems[b])
        pltpu.async_copy(small8.at[pl.ds(off, PK)], r8s[b], sems[b])
        pltpu.async_copy(s0t.at[pl.ds(off, PK)], r64s[b], sems[b])
        pltpu.async_copy(s1t.at[pl.ds(off, PK)], r128s[b], sems[b])

    def drain(b):
        pltpu.make_async_copy(lab.at[pl.ds(0, PK)], ibufs[b], sems[b]).wait()
        pltpu.make_async_copy(small8.at[pl.ds(0, PK)], r8s[b], sems[b]).wait()
        pltpu.make_async_copy(s0t.at[pl.ds(0, PK)], r64s[b], sems[b]).wait()
        pltpu.make_async_copy(s1t.at[pl.ds(0, PK)], r128s[b], sems[b]).wait()

    load(0, 0)

    def pair(i2, carry):
        for b in (0, 1):
            ci = 2 * i2 + b

            @pl.when(ci + 1 < nc)
            def _():
                load(ci + 1, 1 - b)

            drain(b)
            pltpu.sync_copy(r8s[b], a8.at[ibufs[b]], add=True)
            pltpu.sync_copy(r64s[b], a64.at[ibufs[b]], add=True)
            pltpu.sync_copy(r128s[b], a128.at[ibufs[b]], add=True)
        return carry

    lax.fori_loop(0, nc // 2, pair, 0)
    plsc.subcore_barrier()
    pltpu.sync_copy(a8.at[pl.ds(s * zr, zr)],
                    o8.at[pl.ds(c * 4096 + s * zr, zr)])
    pltpu.sync_copy(a64.at[pl.ds(s * zr, zr)],
                    o64.at[pl.ds(c * 4096 + s * zr, zr)])
    pltpu.sync_copy(a128.at[pl.ds(s * zr, zr)],
                    o128.at[pl.ds(c * 4096 + s * zr, zr)])


@functools.cache
def _pool_call():
    f32 = jnp.float32
    return pl.kernel(
        _pool_body,
        out_type=[jax.ShapeDtypeStruct((N, 8), f32),
                  jax.ShapeDtypeStruct((N, 64), f32),
                  jax.ShapeDtypeStruct((N, 128), f32)],
        mesh=_mesh(),
        compiler_params=pltpu.CompilerParams(use_tc_tiling_on_sc=False),
        scratch_types=(
            [pltpu.VMEM((PK,), jnp.int32)] * 2
            + [pltpu.VMEM((PK, 8), f32)] * 2
            + [pltpu.VMEM((PK, 64), f32)] * 2
            + [pltpu.VMEM((PK, 128), f32)] * 2
            + [pltpu.SemaphoreType.DMA] * 2
            + [pltpu.VMEM_SHARED((4096, 8), f32),
               pltpu.VMEM_SHARED((4096, 64), f32),
               pltpu.VMEM_SHARED((4096, 128), f32)]
        ),
    )


NB = 8


def _edge_pass(table, ixf4, out, bufs, acc, c, s, zr, width, rd):
    ibufs, isems, rbufs, rsems = bufs
    _zero_zbuf(rbufs[0], width)
    _zero_acc(rbufs[0], acc, zr, width, s * zr)
    plsc.subcore_barrier()
    nc = E // 32 // PK
    nblk = nc // NB
    tbb = c * (E // 2 // PK // NB) + s * nblk

    pltpu.sync_copy(ixf4.at[tbb], ibufs[0])
    pltpu.async_copy(ixf4.at[tbb + 1], ibufs[1], isems[1])
    for k0 in range(rd - 1):
        pltpu.async_copy(table.at[ibufs[0].at[k0, 0]], rbufs[k0], rsems[k0])

    def pairblock(j2, carry):
        for jj in (0, 1):
            j = 2 * j2 + jj
            ibc, ibn = ibufs[jj], ibufs[1 - jj]
            for k in range(NB):
                ci = j * NB + k
                nk, gslot = k + rd - 1, (k + rd - 1) % rd
                if nk == NB:
                    @pl.when(j < nblk - 1)
                    def _():
                        pltpu.make_async_copy(
                            ixf4.at[tbb], ibn, isems[1 - jj]).wait()
                if nk < NB:
                    @pl.when(ci + rd - 1 < nc)
                    def _(nk=nk, gslot=gslot, ibc=ibc):
                        pltpu.async_copy(table.at[ibc.at[nk, 0]],
                                         rbufs[gslot], rsems[gslot])
                else:
                    @pl.when(ci + rd - 1 < nc)
                    def _(nk=nk - NB, gslot=gslot, ibn=ibn):
                        pltpu.async_copy(table.at[ibn.at[nk, 0]],
                                         rbufs[gslot], rsems[gslot])
                slot = k % rd
                pltpu.make_async_copy(table.at[ibc.at[k, 0]],
                                      rbufs[slot], rsems[slot]).wait()
                pltpu.sync_copy(rbufs[slot], acc.at[ibc.at[k, 1]], add=True)
                if k == NB - 1:
                    @pl.when(j < nblk - 2)
                    def _(ibc=ibc, jj=jj):
                        pltpu.async_copy(ixf4.at[tbb + j + 2], ibc, isems[jj])
        return carry

    lax.fori_loop(0, nblk // 2, pairblock, 0)
    plsc.subcore_barrier()
    pltpu.sync_copy(acc.at[pl.ds(s * zr, zr)], out.at[c, pl.ds(s * zr, zr)])


def _edge_scratch(width, rd):
    f32 = jnp.float32
    return (
        [pltpu.VMEM((NB, 2, PK), jnp.int32)] * 2
        + [pltpu.SemaphoreType.DMA] * 2
        + [pltpu.VMEM((PK, width), f32)] * rd
        + [pltpu.SemaphoreType.DMA] * rd
        + [pltpu.VMEM_SHARED((2 * N, width), f32)]
    )


def _edge0_body(table, ixf4, out, i0, i1, is0, is1,
                r0, r1, rs0, rs1, acc):
    c = lax.axis_index("c")
    s = lax.axis_index("s")
    bufs = ((i0, i1), (is0, is1), (r0, r1), (rs0, rs1))
    _edge_pass(table, ixf4, out, bufs, acc, c, s, (2 * N) // 16, TW0, 2)


@functools.cache
def _edge0_call():
    return pl.kernel(
        _edge0_body,
        out_type=jax.ShapeDtypeStruct((2, 2 * N, TW0), jnp.float32),
        mesh=_mesh(),
        compiler_params=pltpu.CompilerParams(use_tc_tiling_on_sc=False),
        scratch_types=_edge_scratch(TW0, 2),
    )


def _edge1_body(t0, t1, t2, t3, ixf4, o0, o1, o2, o3, i0, i1, is0, is1,
                r0, r1, r2, r3, rs0, rs1, rs2, rs3, acc):
    c = lax.axis_index("c")
    s = lax.axis_index("s")
    bufs = ((i0, i1), (is0, is1), (r0, r1, r2, r3), (rs0, rs1, rs2, rs3))
    for table, out in ((t0, o0), (t1, o1), (t2, o2), (t3, o3)):
        _edge_pass(table, ixf4, out, bufs, acc, c, s, (2 * N) // 16, TW1, 4)


@functools.cache
def _edge1_call():
    return pl.kernel(
        _edge1_body,
        out_type=[jax.ShapeDtypeStruct((2, 2 * N, TW1), jnp.float32)] * 4,
        mesh=_mesh(),
        compiler_params=pltpu.CompilerParams(use_tc_tiling_on_sc=False),
        scratch_types=_edge_scratch(TW1, 4),
    )


BM = 1024
GRID = N // BM


def _bn_relu(y, gamma, beta):
    m = jnp.mean(y, axis=0)
    v = jnp.mean((y - m) ** 2, axis=0)
    return jnp.maximum(gamma * (y - m) / jnp.sqrt(v + EPS) + beta, 0.0)


def _mm(a, w):
    return jnp.dot(a, w, preferred_element_type=jnp.float32)


def _row_spec(c):
    return pl.BlockSpec((BM, c), lambda i: (i, 0))


def _acc_spec(c, row_block_off):
    return pl.BlockSpec((2, BM, c), lambda i: (0, i + row_block_off, 0))


def _full_spec(shape):
    nd = len(shape)
    return pl.BlockSpec(shape, (lambda i: (0,) * nd))


def _tc1_pre_body(p8_ref, p64_ref, w_ref, b_ref, y_ref):
    p8 = p8_ref[...]
    cnt = p8[:, 4:5]
    y_ref[...] = (_mm(p64_ref[...] / cnt, w_ref[...][:64])
                  + _mm(p8[:, 0:4] / cnt, w_ref[...][64:]) + b_ref[...])


def _tc1_bn_body(y_ref, g_ref, be_ref, out_ref):
    x0 = _bn_relu(y_ref[...], g_ref[...], be_ref[...])
    out_ref[...] = jnp.concatenate(
        [x0, jnp.ones((N, 1), jnp.float32), jnp.zeros((N, TW0 - 65), jnp.float32)], axis=1)


def _tc2a_pre_body(ap_ref, an_ref, x0_ref,
                   wpl, wpr, bpr, wnl, wnr, bnr,
                   yp_ref, yn_ref, cnt_ref):
    accp = ap_ref[0] + ap_ref[1]
    accn = an_ref[0] + an_ref[1]
    cp = jnp.maximum(accp[:, 64:65], 1.0)
    cn = jnp.maximum(accn[:, 64:65], 1.0)
    x0 = x0_ref[...][:, :64]
    yp_ref[...] = _mm(accp[:, :64] / cp, wpl[...]) + _mm(x0, wpr[...]) + bpr[...]
    yn_ref[...] = _mm(accn[:, :64] / cn, wnl[...]) + _mm(x0, wnr[...]) + bnr[...]
    cnt_ref[...] = jnp.concatenate([cp, cn], axis=1)


def _tc2a_bn_body(yp_ref, yn_ref, bng, bnb, x1_ref, x2_ref):
    x1_ref[...] = _bn_relu(yp_ref[...], bng[...][:128], bnb[...][:128])
    x2_ref[...] = _bn_relu(yn_ref[...], bng[...][128:], bnb[...][128:])


def _tc2b_pre_body(x1_ref, x2_ref, p8_ref, p128_ref, mpw, mpb, mnw, mnb,
                   yp_ref, yn_ref):
    p8 = p8_ref[...]
    cnt = p8[:, 4:5]
    skip1 = p128_ref[...] / cnt
    coords = p8[:, 0:4] / cnt
    yp_ref[...] = (_mm(x2_ref[...], mpw[...][:128]) + _mm(skip1, mpw[...][128:256])
                   + _mm(coords, mpw[...][256:]) + mpb[...])
    yn_ref[...] = (_mm(x1_ref[...], mnw[...][:128]) + _mm(skip1, mnw[...][128:256])
                   + _mm(coords, mnw[...][256:]) + mnb[...])


def _tc2b_bn_body(yp_ref, yn_ref, mpg, mpbe, mng, mnbe,
                  xp_lo_ref, xp_hi_ref, xn_lo_ref, xn_hi_ref):
    xp = _bn_relu(yp_ref[...], mpg[...], mpbe[...])
    xn = _bn_relu(yn_ref[...], mng[...], mnbe[...])
    xp_lo_ref[...] = xp[:, :64]
    xp_hi_ref[...] = xp[:, 64:]
    xn_lo_ref[...] = xn[:, :64]
    xn_hi_ref[...] = xn[:, 64:]


def _tc3_pre_body(a_lo, a_hi, b_lo, b_hi,
                  cnt_ref, xs_lo, xs_hi, wl, wr, bias, y_ref):
    ca = jnp.maximum(cnt_ref[...][:, 0:1], 1.0)
    cb = jnp.maximum(cnt_ref[...][:, 1:2], 1.0)
    y_ref[...] = (_mm((a_lo[0] + a_lo[1]) / ca, wl[...][0:64])
                  + _mm((a_hi[0] + a_hi[1]) / ca, wl[...][64:128])
                  + _mm((b_lo[0] + b_lo[1]) / cb, wl[...][128:192])
                  + _mm((b_hi[0] + b_hi[1]) / cb, wl[...][192:256])
                  + _mm(xs_lo[...], wr[...][:64]) + _mm(xs_hi[...], wr[...][64:])
                  + bias[...])


def _tc3_bn_body(y1_ref, y2_ref, bng, bnb, z1_ref, z2_ref):
    z1_ref[...] = _bn_relu(y1_ref[...], bng[...][:256], bnb[...][:256])
    z2_ref[...] = _bn_relu(y2_ref[...], bng[...][256:], bnb[...][256:])


def _tc3b_body(z1_ref, z2_ref, lrw, lrb, out_ref):
    out_ref[...] = jnp.maximum(
        _mm(z1_ref[...], lrw[...][:256]) + _mm(z2_ref[...], lrw[...][256:])
        + lrb[...], 0.0)


def kernel(labels, fx, fy, skip0, skip1, edges_nn, params):
    f32 = jnp.float32
    ii = jnp.arange(H, dtype=f32) / (H - 1)
    jj = jnp.arange(W, dtype=f32) / (W - 1)
    xx = jnp.broadcast_to(ii[:, None], (H, W)).reshape(HW, 1)
    yy = jnp.broadcast_to(jj[None, :], (H, W)).reshape(HW, 1)
    coords = jnp.broadcast_to(jnp.concatenate([xx, yy], 1)[None], (B, HW, 2))
    small8 = jnp.concatenate(
        [coords, fx.reshape(B, HW, 1), fy.reshape(B, HW, 1),
         jnp.ones((B, HW, 1), f32), jnp.zeros((B, HW, 3), f32)],
        axis=2).reshape(B * HW, 8)
    s0t = skip0.reshape(B, 64, HW).transpose(0, 2, 1).reshape(B * HW, 64)
    s1t = skip1.reshape(B, 128, HW).transpose(0, 2, 1).reshape(B * HW, 128)
    b_half = (jnp.arange(B * HW, dtype=jnp.int32) // HW) & 1
    lab = labels.reshape(B * HW) + b_half * S
    flat = edges_nn[1] + jnp.where(edges_nn[2] < 0, N, 0).astype(jnp.int32)
    ixf = jnp.stack([edges_nn[0].reshape(E // PK, PK),
                     flat.reshape(E // PK, PK)],
                    axis=1).reshape(E // PK // NB, NB, 2, PK)

    p8, p64, p128 = _pool_call()(small8, s0t, s1t, lab)

    p = params['pre_merger']
    y0 = pl.pallas_call(
        _tc1_pre_body,
        grid=(GRID,),
        in_specs=[_row_spec(8), _row_spec(64),
                  _full_spec((68, 64)), _full_spec((64,))],
        out_specs=_row_spec(64),
        out_shape=jax.ShapeDtypeStruct((N, 64), f32),
    )(p8, p64, p['W'], p['b'])
    x0aug = pl.pallas_call(
        _tc1_bn_body, out_shape=jax.ShapeDtypeStruct((N, TW0), f32),
    )(y0, p['gamma'], p['beta'])

    eacc0 = _edge0_call()(x0aug, ixf)

    g = params['gcn0']
    yp, yn, cnt = pl.pallas_call(
        _tc2a_pre_body,
        grid=(GRID,),
        in_specs=[_acc_spec(TW0, 0), _acc_spec(TW0, GRID), _row_spec(TW0)]
        + [_full_spec(s) for s in ((64, 128), (64, 128), (128,),
                                   (64, 128), (64, 128), (128,))],
        out_specs=[_row_spec(128), _row_spec(128), _row_spec(2)],
        out_shape=[jax.ShapeDtypeStruct((N, 128), f32)] * 2
        + [jax.ShapeDtypeStruct((N, 2), f32)],
    )(eacc0, eacc0, x0aug,
      g['Wpl'], g['Wpr'], g['bpr'], g['Wnl'], g['Wnr'], g['bnr'])
    x1, x2 = pl.pallas_call(
        _tc2a_bn_body,
        out_shape=[jax.ShapeDtypeStruct((N, 128), f32)] * 2,
    )(yp, yn, g['bn_gamma'], g['bn_beta'])

    mp = params['merger_pos0']
    mn = params['merger_neg0']
    yp2, yn2 = pl.pallas_call(
        _tc2b_pre_body,
        grid=(GRID,),
        in_specs=[_row_spec(128), _row_spec(128), _row_spec(8), _row_spec(128)]
        + [_full_spec(s) for s in ((260, 128), (128,), (260, 128), (128,))],
        out_specs=[_row_spec(128), _row_spec(128)],
        out_shape=[jax.ShapeDtypeStruct((N, 128), f32)] * 2,
    )(x1, x2, p8, p128, mp['W'], mp['b'], mn['W'], mn['b'])
    xp_lo, xp_hi, xn_lo, xn_hi = pl.pallas_call(
        _tc2b_bn_body,
        out_shape=[jax.ShapeDtypeStruct((N, 64), f32)] * 4,
    )(yp2, yn2, mp['gamma'], mp['beta'], mn['gamma'], mn['beta'])

    e1 = _edge1_call()(xp_lo, xp_hi, xn_lo, xn_hi, ixf)
    exp_lo, exp_hi, exn_lo, exn_hi = e1

    g1 = params['gcn1']

    def tc3_pre(a_lo, a_hi, b_lo, b_hi, xs_lo, xs_hi, wl, wr, bias):
        return pl.pallas_call(
            _tc3_pre_body,
            grid=(GRID,),
            in_specs=[_acc_spec(TW1, 0), _acc_spec(TW1, 0),
                      _acc_spec(TW1, GRID), _acc_spec(TW1, GRID), _row_spec(2)]
            + [_row_spec(TW1)] * 2
            + [_full_spec(s) for s in ((256, 256), (128, 256), (256,))],
            out_specs=_row_spec(256),
            out_shape=jax.ShapeDtypeStruct((N, 256), f32),
        )(a_lo, a_hi, b_lo, b_hi, cnt, xs_lo, xs_hi, wl, wr, bias)

    y1 = tc3_pre(exp_lo, exp_hi, exn_lo, exn_hi, xp_lo, xp_hi,
                 g1['Wpl'], g1['Wpr'], g1['bpr'])
    y2 = tc3_pre(exn_lo, exn_hi, exp_lo, exp_hi, xn_lo, xn_hi,
                 g1['Wnl'], g1['Wnr'], g1['bnr'])
    z1, z2 = pl.pallas_call(
        _tc3_bn_body,
        out_shape=[jax.ShapeDtypeStruct((N, 256), f32)] * 2,
    )(y1, y2, g1['bn_gamma'], g1['bn_beta'])

    lr = params['lin_reduc']
    return pl.pallas_call(
        _tc3b_body, out_shape=jax.ShapeDtypeStruct((N, 256), f32),
    )(z1, z2, lr['W'], lr['b'])
